# baseline (speedup 1.0000x reference)
import functools

import jax
import jax.numpy as jnp
from jax import lax
from jax.experimental import pallas as pl
from jax.experimental.pallas import tpu as pltpu
from jax.experimental.pallas import tpu_sc as plsc

VOCAB = 100000
D = 64
C = 50
B = 4096
NC = 2
NS = 16
NW = NC * NS
BPW = B // NW
LANES = 16

VT = 512


def _sc_pool(x_r, emb_table):
  mesh = plsc.VectorSubcoreMesh(core_axis_name="c", subcore_axis_name="s")

  @functools.partial(
      pl.kernel,
      mesh=mesh,
      out_type=jax.ShapeDtypeStruct((B, D), jnp.float32),
      compiler_params=pltpu.CompilerParams(use_tc_tiling_on_sc=False),
      scratch_types=[
          pltpu.VMEM((BPW, C), jnp.int32),
          pltpu.VMEM((C, D), jnp.float32),
          pltpu.VMEM((BPW, D), jnp.float32),
          pltpu.SemaphoreType.DMA,
      ],
  )
  def k(x_hbm, table_hbm, out_hbm, idx_v, rows_v, out_v, sem):
    w = lax.axis_index("s") * NC + lax.axis_index("c")
    pltpu.sync_copy(x_hbm.at[w], idx_v)
    scale = jnp.float32(1.0 / C)

    def body(g, carry):
      pltpu.async_copy(table_hbm.at[idx_v.at[g]], rows_v, sem).wait()
      accs = [jnp.zeros((LANES,), jnp.float32) for _ in range(2 * (D // LANES))]
      for j in range(C):
        p = j % 2
        for kk in range(D // LANES):
          accs[p * (D // LANES) + kk] = (
              accs[p * (D // LANES) + kk]
              + rows_v[j, pl.ds(kk * LANES, LANES)])
      for kk in range(D // LANES):
        out_v[g, pl.ds(kk * LANES, LANES)] = (
            accs[kk] + accs[(D // LANES) + kk]) * scale
      return carry

    lax.fori_loop(0, BPW, body, 0)
    pltpu.sync_copy(out_v, out_hbm.at[pl.ds(w * BPW, BPW)])

  return k(x_r, emb_table)


def _tc_matmul(cbow, fc_w, fc_b2):
  nv = pl.cdiv(VOCAB, VT)

  def mm(cbow_ref, w_ref, b_ref, out_ref):
    out_ref[...] = lax.dot_general(
        cbow_ref[...], w_ref[...],
        (((1,), (1,)), ((), ())),
        preferred_element_type=jnp.float32) + b_ref[...]

  return pl.pallas_call(
      mm,
      grid=(nv,),
      in_specs=[
          pl.BlockSpec((B, D), lambda v: (0, 0)),
          pl.BlockSpec((VT, D), lambda v: (v, 0)),
          pl.BlockSpec((1, VT), lambda v: (0, v)),
      ],
      out_specs=pl.BlockSpec((B, VT), lambda v: (0, v)),
      out_shape=jax.ShapeDtypeStruct((B, VOCAB), jnp.float32),
  )(cbow, fc_w, fc_b2)


def kernel(x, emb_table, fc_w, fc_b):
  x_r = x.astype(jnp.int32).reshape(NW, BPW, C)
  cbow = _sc_pool(x_r, emb_table)
  return _tc_matmul(cbow, fc_w, fc_b.reshape(1, VOCAB))


# speedup vs baseline: 1.0024x; 1.0024x over previous
import functools

import jax
import jax.numpy as jnp
from jax import lax
from jax.experimental import pallas as pl
from jax.experimental.pallas import tpu as pltpu
from jax.experimental.pallas import tpu_sc as plsc

VOCAB = 100000
D = 64
C = 50
B = 4096
NC = 2
NS = 16
NW = NC * NS
BPW = B // NW
LANES = 16

VT = 1024


def _sc_pool(x_r, emb_table):
  mesh = plsc.VectorSubcoreMesh(core_axis_name="c", subcore_axis_name="s")

  @functools.partial(
      pl.kernel,
      mesh=mesh,
      out_type=jax.ShapeDtypeStruct((B, D), jnp.float32),
      compiler_params=pltpu.CompilerParams(use_tc_tiling_on_sc=False),
      scratch_types=[
          pltpu.VMEM((BPW, C), jnp.int32),
          pltpu.VMEM((C, D), jnp.float32),
          pltpu.VMEM((BPW, D), jnp.float32),
          pltpu.SemaphoreType.DMA,
      ],
  )
  def k(x_hbm, table_hbm, out_hbm, idx_v, rows_v, out_v, sem):
    w = lax.axis_index("s") * NC + lax.axis_index("c")
    pltpu.sync_copy(x_hbm.at[w], idx_v)
    scale = jnp.float32(1.0 / C)

    def body(g, carry):
      pltpu.async_copy(table_hbm.at[idx_v.at[g]], rows_v, sem).wait()
      accs = [jnp.zeros((LANES,), jnp.float32) for _ in range(2 * (D // LANES))]
      for j in range(C):
        p = j % 2
        for kk in range(D // LANES):
          accs[p * (D // LANES) + kk] = (
              accs[p * (D // LANES) + kk]
              + rows_v[j, pl.ds(kk * LANES, LANES)])
      for kk in range(D // LANES):
        out_v[g, pl.ds(kk * LANES, LANES)] = (
            accs[kk] + accs[(D // LANES) + kk]) * scale
      return carry

    lax.fori_loop(0, BPW, body, 0)
    pltpu.sync_copy(out_v, out_hbm.at[pl.ds(w * BPW, BPW)])

  return k(x_r, emb_table)


def _tc_matmul(cbow, fc_w, fc_b2):
  nv = pl.cdiv(VOCAB, VT)

  def mm(cbow_ref, w_ref, b_ref, out_ref):
    out_ref[...] = lax.dot_general(
        cbow_ref[...], w_ref[...],
        (((1,), (1,)), ((), ())),
        preferred_element_type=jnp.float32) + b_ref[...]

  return pl.pallas_call(
      mm,
      grid=(nv,),
      in_specs=[
          pl.BlockSpec((B, D), lambda v: (0, 0)),
          pl.BlockSpec((VT, D), lambda v: (v, 0)),
          pl.BlockSpec((1, VT), lambda v: (0, v)),
      ],
      out_specs=pl.BlockSpec((B, VT), lambda v: (0, v)),
      out_shape=jax.ShapeDtypeStruct((B, VOCAB), jnp.float32),
  )(cbow, fc_w, fc_b2)


def kernel(x, emb_table, fc_w, fc_b):
  x_r = x.astype(jnp.int32).reshape(NW, BPW, C)
  cbow = _sc_pool(x_r, emb_table)
  return _tc_matmul(cbow, fc_w, fc_b.reshape(1, VOCAB))


# speedup vs baseline: 1.0028x; 1.0004x over previous
import functools

import jax
import jax.numpy as jnp
from jax import lax
from jax.experimental import pallas as pl
from jax.experimental.pallas import tpu as pltpu
from jax.experimental.pallas import tpu_sc as plsc

VOCAB = 100000
D = 64
C = 50
B = 4096
NC = 2
NS = 16
NW = NC * NS
BPW = B // NW
LANES = 16

VT = 1024


def _sc_pool(x_r, emb_table):
  mesh = plsc.VectorSubcoreMesh(core_axis_name="c", subcore_axis_name="s")

  @functools.partial(
      pl.kernel,
      mesh=mesh,
      out_type=jax.ShapeDtypeStruct((B, D), jnp.float32),
      compiler_params=pltpu.CompilerParams(use_tc_tiling_on_sc=False),
      scratch_types=[
          pltpu.VMEM((BPW, C), jnp.int32),
          pltpu.VMEM((C, D), jnp.float32),
          pltpu.VMEM((BPW, D), jnp.float32),
          pltpu.SemaphoreType.DMA,
      ],
  )
  def k(x_hbm, table_hbm, out_hbm, idx_v, rows_v, out_v, sem):
    w = lax.axis_index("s") * NC + lax.axis_index("c")
    pltpu.sync_copy(x_hbm.at[w], idx_v)
    scale = jnp.float32(1.0 / C)

    def body(g, carry):
      pltpu.async_copy(table_hbm.at[idx_v.at[g]], rows_v, sem).wait()
      accs = [jnp.zeros((LANES,), jnp.float32) for _ in range(2 * (D // LANES))]
      for j in range(C):
        p = j % 2
        for kk in range(D // LANES):
          accs[p * (D // LANES) + kk] = (
              accs[p * (D // LANES) + kk]
              + rows_v[j, pl.ds(kk * LANES, LANES)])
      for kk in range(D // LANES):
        out_v[g, pl.ds(kk * LANES, LANES)] = (
            accs[kk] + accs[(D // LANES) + kk]) * scale
      return carry

    lax.fori_loop(0, BPW, body, 0)
    pltpu.sync_copy(out_v, out_hbm.at[pl.ds(w * BPW, BPW)])

  return k(x_r, emb_table)


def _tc_matmul(cbow, fc_w, fc_b2):
  nv = pl.cdiv(VOCAB, VT)

  def mm(cbow_ref, w_ref, b_ref, out_ref):
    out_ref[...] = lax.dot_general(
        cbow_ref[...], w_ref[...],
        (((1,), (1,)), ((), ())),
        preferred_element_type=jnp.float32) + b_ref[...]

  return pl.pallas_call(
      mm,
      grid=(nv,),
      in_specs=[
          pl.BlockSpec((B, D), lambda v: (0, 0)),
          pl.BlockSpec((VT, D), lambda v: (v, 0)),
          pl.BlockSpec((1, VT), lambda v: (0, v)),
      ],
      out_specs=pl.BlockSpec((B, VT), lambda v: (0, v)),
      out_shape=jax.ShapeDtypeStruct((B, VOCAB), jnp.float32),
      compiler_params=pltpu.CompilerParams(
          vmem_limit_bytes=128 * 1024 * 1024),
  )(cbow, fc_w, fc_b2)


def kernel(x, emb_table, fc_w, fc_b):
  x_r = x.astype(jnp.int32).reshape(NW, BPW, C)
  cbow = _sc_pool(x_r, emb_table)
  return _tc_matmul(cbow, fc_w, fc_b.reshape(1, VOCAB))
